# Initial kernel scaffold; baseline (speedup 1.0000x reference)
import jax
import jax.numpy as jnp
from jax.experimental import pallas as pl


def kernel(x, codebook):
    raise NotImplementedError("write your pallas kernel here")


# speedup vs baseline: 1.0307x; 1.0307x over previous
import jax
import jax.numpy as jnp
from jax import lax
from jax.experimental import pallas as pl

_TM = 256


def _vq_argmin_kernel(x_ref, cb_ref, xn_ref, cn_ref, out_ref):
    x_tile = x_ref[...]
    cb = cb_ref[...]
    scores_t = lax.dot_general(
        cb, x_tile, (((1,), (1,)), ((), ())),
        preferred_element_type=jnp.float32)
    d = (xn_ref[...] + cn_ref[...]) - 2.0 * scores_t
    m = jnp.min(d, axis=0, keepdims=True)
    iota = lax.broadcasted_iota(jnp.int32, d.shape, 0)
    idx = jnp.min(jnp.where(d <= m, iota, 2 ** 30), axis=0)
    out_ref[0, 0, :] = idx


def kernel(x, codebook):
    b, t, c, h, w = x.shape
    n = b * t * h * w
    xf = jnp.transpose(x, (0, 1, 3, 4, 2)).reshape(n, c)
    xnorm = jnp.sum(xf ** 2, axis=1, keepdims=True).reshape(1, n)
    cnorm = jnp.sum(codebook ** 2, axis=1).reshape(-1, 1)
    num_tiles = n // _TM
    out = pl.pallas_call(
        _vq_argmin_kernel,
        grid=(num_tiles,),
        in_specs=[
            pl.BlockSpec((_TM, c), lambda i: (i, 0)),
            pl.BlockSpec(codebook.shape, lambda i: (0, 0)),
            pl.BlockSpec((1, _TM), lambda i: (0, i)),
            pl.BlockSpec((codebook.shape[0], 1), lambda i: (0, 0)),
        ],
        out_specs=pl.BlockSpec((1, 1, _TM), lambda i: (i, 0, 0)),
        out_shape=jax.ShapeDtypeStruct((num_tiles, 1, _TM), jnp.int32),
    )(xf, codebook, xnorm, cnorm)
    return out.reshape(b, t, h, w)
